# Initial kernel scaffold; baseline (speedup 1.0000x reference)
import jax
import jax.numpy as jnp
from jax.experimental import pallas as pl


def kernel(camera, label):
    raise NotImplementedError("write your pallas kernel here")


# speedup vs baseline: 5.4642x; 5.4642x over previous
import functools

import jax
import jax.numpy as jnp
import numpy as np
from jax.experimental import pallas as pl
from jax.experimental.pallas import tpu as pltpu

_K = 16
_C = 4
_BETA = 500.0
_EPS_SM = 1e-12
_EPS = 1e-07
_HALF_INV_BW_CAM = 100.0
_HALF_INV_BW_LAB = 500.0

_HB = 16


def _hist_kernel(cam_ref, lab_ref, out_ref):
    j = pl.program_id(0)

    cam = cam_ref[...]
    lab = lab_ref[...]
    b, _, hb, w = cam.shape

    gray = (cam[:, 0] + cam[:, 1] + cam[:, 2]) * (1.0 / 3.0)

    cam_edges = jax.lax.broadcasted_iota(
        jnp.int32, (_K + 1, 1, 1, 1), 0).astype(jnp.float32) * (_HALF_INV_BW_CAM / _K)
    gscaled = gray * _HALF_INV_BW_CAM
    t_cam = jnp.tanh(gscaled[None] - cam_edges)
    t_cam = jnp.sum(t_cam, axis=1)
    a = 0.5 * (t_cam[:_K] - t_cam[1:])

    m = jnp.max(lab, axis=1, keepdims=True)
    e = jnp.exp((lab - m) * _BETA)
    idx = jax.lax.broadcasted_iota(jnp.int32, (1, _C, 1, 1), 1).astype(jnp.float32)
    num = jnp.sum(e * idx, axis=1)
    den = jnp.sum(e, axis=1) + _EPS_SM
    amax = num / den

    lab_edges = (jax.lax.broadcasted_iota(
        jnp.int32, (_C + 1, 1, 1, 1), 0).astype(jnp.float32) - 0.5) * _HALF_INV_BW_LAB
    t_lab = jnp.tanh(amax[None] * _HALF_INV_BW_LAB - lab_edges)
    t_lab = jnp.sum(t_lab, axis=1)
    l = 0.5 * (t_lab[:_C] - t_lab[1:])

    ones = jnp.ones((1, hb, w), dtype=jnp.float32)
    a_aug = jnp.concatenate([a, ones], axis=0)
    l_aug = jnp.concatenate([l, ones], axis=0)

    m_blk = jnp.zeros((_K + 1, _C + 1), dtype=jnp.float32)
    for hh in range(hb):
        m_blk = m_blk + jax.lax.dot_general(
            a_aug[:, hh, :], l_aug[:, hh, :], (((1,), (1,)), ((), ())),
            preferred_element_type=jnp.float32,
            precision=jax.lax.Precision.HIGHEST,
        )

    @pl.when(j == 0)
    def _():
        out_ref[...] = jnp.zeros_like(out_ref)

    out_ref[...] += m_blk


def _nid_kernel(m_ref, out_ref, *, norm):
    m = m_ref[...]
    p_cl = m[:_K, :_C] / norm
    p_c = m[:_K, _C:_C + 1] / norm
    p_l = m[_K:_K + 1, :_C] / norm

    p_cl = p_cl / jnp.sum(p_cl)
    p_c = p_c / jnp.sum(p_c)
    p_l = p_l / jnp.sum(p_l)

    outer = p_c * p_l
    log_pcl = jnp.log(p_cl + _EPS)
    mi = jnp.sum(p_cl * (log_pcl - jnp.log(outer + _EPS)))
    h_ent = -jnp.sum(p_cl * log_pcl)
    nid = 1.0 - mi / h_ent
    out_ref[...] = jnp.full((1, 1), (nid - 0.95) * 20.0, dtype=jnp.float32)


@jax.jit
def kernel(camera, label):
    b, _, h, w = camera.shape
    n_strips = h // _HB

    partials = pl.pallas_call(
        _hist_kernel,
        grid=(n_strips,),
        in_specs=[
            pl.BlockSpec((b, 3, _HB, w), lambda j: (0, 0, j, 0)),
            pl.BlockSpec((b, _C, _HB, w), lambda j: (0, 0, j, 0)),
        ],
        out_specs=pl.BlockSpec((_K + 1, _C + 1), lambda j: (0, 0)),
        out_shape=jax.ShapeDtypeStruct((_K + 1, _C + 1), jnp.float32),
        compiler_params=pltpu.CompilerParams(
            dimension_semantics=("arbitrary",),
        ),
    )(camera, label)

    norm = float(b * h * w)
    out = pl.pallas_call(
        functools.partial(_nid_kernel, norm=norm),
        out_shape=jax.ShapeDtypeStruct((1, 1), jnp.float32),
    )(partials)
    return out[0, 0]
